# baseline (speedup 1.0000x reference)
import functools

import jax
import jax.numpy as jnp
from jax import lax
from jax.experimental import pallas as pl
from jax.experimental.pallas import tpu as pltpu
from jax.experimental.pallas import tpu_sc as plsc

B, T, H, C = 8, 2048, 128, 32
ENC = 1024
TM = T - ENC

NC, NS = 2, 16
NW = NC * NS
ROWS_PER_W = (B * C) // NW
QT = T // 128
QM = TM // 128


def _sc_gather_body(shuffle_hbm, spikes_hbm, out_hbm, idx_v, plane_v, res_v, sem):
    wid = lax.axis_index("s") * NC + lax.axis_index("c")
    r0 = wid * ROWS_PER_W
    pltpu.sync_copy(shuffle_hbm.at[pl.ds(ENC, TM)], idx_v)
    cps = []
    for r in range(ROWS_PER_W):
        fr = r0 + r
        b = fr // C
        c = fr % C
        cps.append(pltpu.async_copy(
            spikes_hbm.at[b].at[c // 8].at[:, c % 8],
            plane_v.at[r], sem))
    for cp in cps:
        cp.wait()
    for g in range(TM // 16):
        t16 = idx_v[pl.ds(g * 16, 16)]
        q16 = lax.shift_right_logical(t16, 7)
        l16 = lax.bitwise_and(t16, 127)
        for r in range(ROWS_PER_W):
            v16 = plsc.load_gather(plane_v, [jnp.full((16,), r, jnp.int32), q16, l16])
            res_v[r, pl.ds(g * 16, 16)] = v16
    pltpu.sync_copy(res_v, out_hbm.at[pl.ds(r0, ROWS_PER_W)])


_sc_gather = functools.partial(
    pl.kernel,
    mesh=plsc.VectorSubcoreMesh(core_axis_name="c", subcore_axis_name="s"),
    out_type=jax.ShapeDtypeStruct((B * C, TM), jnp.int32),
    scratch_types=[
        pltpu.VMEM((TM,), jnp.int32),
        pltpu.VMEM((ROWS_PER_W, QT, 128), jnp.int32),
        pltpu.VMEM((ROWS_PER_W, TM), jnp.int32),
        pltpu.SemaphoreType.DMA,
    ],
    compiler_params=pltpu.CompilerParams(use_tc_tiling_on_sc=False,
                                         needs_layout_passes=False),
)(_sc_gather_body)


def _tc_loss_body(lengths_ref, tokpos_ref, bf_ref, tgt_ref,
                  w1_ref, b1_ref, w2_ref, b2_ref, out_ref, acc_ref):
    b = pl.program_id(0)
    x = bf_ref[0]
    h_t = lax.dot_general(w1_ref[...], x, (((0,), (1,)), ((), ())),
                          preferred_element_type=jnp.float32) + b1_ref[...]
    h_t = jax.nn.gelu(h_t)
    lr_t = lax.dot_general(w2_ref[...], h_t, (((0,), (0,)), ((), ())),
                           preferred_element_type=jnp.float32) + b2_ref[...]
    tgt_t = tgt_ref[0].astype(jnp.float32)
    loss = jnp.exp(lr_t) - tgt_t * lr_t
    mask = tokpos_ref[...] < lengths_ref[b]
    loss = jnp.where(mask, loss, 0.0)

    @pl.when(b == 0)
    def _():
        acc_ref[0] = 0.0
        acc_ref[1] = 0.0

    acc_ref[0] += jnp.sum(loss)
    acc_ref[1] += jnp.sum(mask.astype(jnp.float32))

    @pl.when(b == B - 1)
    def _():
        denom = jnp.maximum(acc_ref[1] * C, 1.0)
        out_ref[0, 0] = acc_ref[0] / denom


_tc_loss = pl.pallas_call(
    _tc_loss_body,
    grid=(B,),
    in_specs=[
        pl.BlockSpec(memory_space=pltpu.SMEM),
        pl.BlockSpec((1, TM), lambda b: (0, 0)),
        pl.BlockSpec((1, TM, H), lambda b: (b, 0, 0)),
        pl.BlockSpec((1, C, TM), lambda b: (b, 0, 0)),
        pl.BlockSpec((H, H), lambda b: (0, 0)),
        pl.BlockSpec((H, 1), lambda b: (0, 0)),
        pl.BlockSpec((H, C), lambda b: (0, 0)),
        pl.BlockSpec((C, 1), lambda b: (0, 0)),
    ],
    out_specs=pl.BlockSpec(memory_space=pltpu.SMEM),
    out_shape=jax.ShapeDtypeStruct((1, 1), jnp.float32),
    scratch_shapes=[pltpu.SMEM((2,), jnp.float32)],
)


def kernel(backbone_features, spikes, shuffle, lengths, encoder_frac, W1, b1, W2, b2):
    del encoder_frac
    spikes_v = (spikes.reshape(B, T // 128, 128, C // 8, 8)
                .transpose(0, 3, 1, 4, 2))
    tgt_t = _sc_gather(shuffle, spikes_v).reshape(B, C, TM)
    tokpos = shuffle[ENC:].reshape(1, TM)
    out = _tc_loss(lengths, tokpos, backbone_features, tgt_t,
                   W1, b1.reshape(H, 1), W2, b2.reshape(C, 1))
    return out[0, 0]


# speedup vs baseline: 1.0797x; 1.0797x over previous
import functools

import jax
import jax.numpy as jnp
from jax import lax
from jax.experimental import pallas as pl
from jax.experimental.pallas import tpu as pltpu
from jax.experimental.pallas import tpu_sc as plsc

B, T, H, C = 8, 2048, 128, 32
ENC = 1024
TM = T - ENC

NC, NS = 2, 16
NW = NC * NS
ROWS_PER_W = (B * C) // NW
QT = T // 128
QM = TM // 128


def _sc_gather_body(shuffle_hbm, spikes_hbm, out_hbm, idx_v, plane_v, res_v, sem):
    wid = lax.axis_index("s") * NC + lax.axis_index("c")
    r0 = wid * ROWS_PER_W
    pltpu.sync_copy(shuffle_hbm.at[pl.ds(ENC, TM)], idx_v)
    cps = []
    for r in range(ROWS_PER_W):
        fr = r0 + r
        b = fr // C
        c = fr % C
        cps.append(pltpu.async_copy(
            spikes_hbm.at[b].at[c // 8].at[:, c % 8],
            plane_v.at[r], sem))
    for cp in cps:
        cp.wait()
    for g in range(TM // 16):
        t16 = idx_v[pl.ds(g * 16, 16)]
        q16 = lax.shift_right_logical(t16, 7)
        l16 = lax.bitwise_and(t16, 127)
        for r in range(ROWS_PER_W):
            v16 = plsc.load_gather(plane_v, [jnp.full((16,), r, jnp.int32), q16, l16])
            res_v[r, pl.ds(g * 16, 16)] = v16
    pltpu.sync_copy(res_v, out_hbm.at[pl.ds(r0, ROWS_PER_W)])


_sc_gather = functools.partial(
    pl.kernel,
    mesh=plsc.VectorSubcoreMesh(core_axis_name="c", subcore_axis_name="s"),
    out_type=jax.ShapeDtypeStruct((B * C, TM), jnp.int32),
    scratch_types=[
        pltpu.VMEM((TM,), jnp.int32),
        pltpu.VMEM((ROWS_PER_W, QT, 128), jnp.int32),
        pltpu.VMEM((ROWS_PER_W, TM), jnp.int32),
        pltpu.SemaphoreType.DMA,
    ],
    compiler_params=pltpu.CompilerParams(use_tc_tiling_on_sc=False,
                                         needs_layout_passes=False),
)(_sc_gather_body)


def _tc_mlp_body(bf_ref, w1_ref, b1_ref, w2_ref, b2_ref, lr_ref):
    x = bf_ref[0]
    h_t = lax.dot_general(w1_ref[...], x, (((0,), (1,)), ((), ())),
                          preferred_element_type=jnp.float32) + b1_ref[...]
    h_t = jax.nn.gelu(h_t)
    lr_ref[0] = lax.dot_general(w2_ref[...], h_t, (((0,), (0,)), ((), ())),
                                preferred_element_type=jnp.float32) + b2_ref[...]


_tc_mlp = pl.pallas_call(
    _tc_mlp_body,
    grid=(B,),
    in_specs=[
        pl.BlockSpec((1, TM, H), lambda b: (b, 0, 0)),
        pl.BlockSpec((H, H), lambda b: (0, 0)),
        pl.BlockSpec((H, 1), lambda b: (0, 0)),
        pl.BlockSpec((H, C), lambda b: (0, 0)),
        pl.BlockSpec((C, 1), lambda b: (0, 0)),
    ],
    out_specs=pl.BlockSpec((1, C, TM), lambda b: (b, 0, 0)),
    out_shape=jax.ShapeDtypeStruct((B, C, TM), jnp.float32),
)


def _tc_loss_body(lengths_ref, tokpos_ref, lr_ref, tgt_ref, out_ref, acc_ref):
    b = pl.program_id(0)
    lr_t = lr_ref[0]
    tgt_t = tgt_ref[0].astype(jnp.float32)
    loss = jnp.exp(lr_t) - tgt_t * lr_t
    mask = tokpos_ref[...] < lengths_ref[b]
    loss = jnp.where(mask, loss, 0.0)

    @pl.when(b == 0)
    def _():
        acc_ref[0] = 0.0
        acc_ref[1] = 0.0

    acc_ref[0] += jnp.sum(loss)
    acc_ref[1] += jnp.sum(mask.astype(jnp.float32))

    @pl.when(b == B - 1)
    def _():
        denom = jnp.maximum(acc_ref[1] * C, 1.0)
        out_ref[0, 0] = acc_ref[0] / denom


_tc_loss = pl.pallas_call(
    _tc_loss_body,
    grid=(B,),
    in_specs=[
        pl.BlockSpec(memory_space=pltpu.SMEM),
        pl.BlockSpec((1, TM), lambda b: (0, 0)),
        pl.BlockSpec((1, C, TM), lambda b: (b, 0, 0)),
        pl.BlockSpec((1, C, TM), lambda b: (b, 0, 0)),
    ],
    out_specs=pl.BlockSpec(memory_space=pltpu.SMEM),
    out_shape=jax.ShapeDtypeStruct((1, 1), jnp.float32),
    scratch_shapes=[pltpu.SMEM((2,), jnp.float32)],
)


def kernel(backbone_features, spikes, shuffle, lengths, encoder_frac, W1, b1, W2, b2):
    del encoder_frac
    spikes_v = (spikes.reshape(B, T // 128, 128, C // 8, 8)
                .transpose(0, 3, 1, 4, 2))
    tgt_t = _sc_gather(shuffle, spikes_v).reshape(B, C, TM)
    lr_t = _tc_mlp(backbone_features, W1, b1.reshape(H, 1), W2, b2.reshape(C, 1))
    tokpos = shuffle[ENC:].reshape(1, TM)
    out = _tc_loss(lengths, tokpos, lr_t, tgt_t)
    return out[0, 0]
